# baseline (speedup 1.0000x reference)
import functools

import jax
import jax.numpy as jnp
from jax import lax
from jax.experimental import pallas as pl
from jax.experimental.pallas import tpu as pltpu
from jax.experimental.pallas import tpu_sc as plsc

N = 10000
E = 320000
F = 128
HD = 256
C = 40
L3W = 48

NSC = 2
NT = 16
CHUNK = 128

NP = N + 112
RPT = NP // NT

E_PAD = 327680
KF = E_PAD // (NT * CHUNK)
KE = E_PAD // (NSC * NT * CHUNK)
IB = 8
HC = 64
NB = 6

_MESH = plsc.VectorSubcoreMesh(core_axis_name="c", subcore_axis_name="s")
_SC_PARAMS = pltpu.CompilerParams(use_tc_tiling_on_sc=False)


def _deg_body(dst_hbm, zeros_hbm, ones_hbm, out_hbm, acc, dst_v, ones_v, sem):
    c = lax.axis_index("c")
    s = lax.axis_index("s")
    r0 = s * RPT
    pltpu.sync_copy(zeros_hbm.at[pl.ds(r0, RPT), pl.ds(0, 8)],
                    acc.at[pl.ds(r0, RPT)])
    pltpu.sync_copy(ones_hbm.at[:, pl.ds(0, 8)], ones_v)
    pltpu.sync_copy(dst_hbm.at[c, s], dst_v)
    plsc.subcore_barrier()

    def step(j, carry):
        pltpu.sync_copy(ones_v, acc.at[dst_v.at[j]], add=True)
        return carry

    lax.fori_loop(0, KE, step, 0)
    plsc.subcore_barrier()
    pltpu.sync_copy(acc.at[pl.ds(r0, RPT)],
                    out_hbm.at[c, pl.ds(r0, RPT), pl.ds(0, 8)])


_deg_kernel = functools.partial(
    pl.kernel,
    out_type=jax.ShapeDtypeStruct((NSC, NP, 128), jnp.float32),
    mesh=_MESH,
    scratch_types=[
        pltpu.VMEM_SHARED((NP, 8), jnp.float32),
        pltpu.VMEM((KE, CHUNK), jnp.int32),
        pltpu.VMEM((CHUNK, 8), jnp.float32),
        pltpu.SemaphoreType.DMA,
    ],
    compiler_params=_SC_PARAMS,
)(_deg_body)


def _agg_blocks(n_blocks, load_idx, h_hbm, acc, src_v, dst_v, bufs, sems):
    nhc = 2 * IB

    def block(b, carry):
        load_idx(b)

        def issue(k):
            j, p = divmod(k, 2)
            return pltpu.async_copy(
                h_hbm.at[src_v.at[j, pl.ds(p * HC, HC)]],
                bufs[k % NB], sems[k % NB])

        descs = [None] * nhc
        for k in range(NB - 1):
            descs[k] = issue(k)
        for k in range(nhc):
            descs[k].wait()
            if k + NB - 1 < nhc:
                descs[k + NB - 1] = issue(k + NB - 1)
            j, p = divmod(k, 2)
            pltpu.sync_copy(bufs[k % NB],
                            acc.at[dst_v.at[j, pl.ds(p * HC, HC)]], add=True)
        return carry

    lax.fori_loop(0, n_blocks, block, 0)


def _aggf_body(h_hbm, src_hbm, dstf_hbm, out_hbm, h_buf, acc, src_v, dst_v,
               b0, b1, b2, b3, b4, b5, s0, s1, s2, s3, s4, s5):
    c = lax.axis_index("c")
    s = lax.axis_index("s")
    r0 = s * RPT

    def load_idx(b):
        pltpu.sync_copy(src_hbm.at[s, pl.ds(b * IB, IB)], src_v)
        pltpu.sync_copy(dstf_hbm.at[s, pl.ds(b * IB, IB)], dst_v)

    for qq in range(2):
        lo = qq * 64
        pltpu.sync_copy(h_hbm.at[pl.ds(c * NP + r0, RPT), pl.ds(lo, 64)],
                        h_buf.at[pl.ds(r0, RPT)])
        pltpu.sync_copy(h_hbm.at[pl.ds(c * NP + r0, RPT), pl.ds(lo, 64)],
                        acc.at[pl.ds(r0, RPT)])
        plsc.subcore_barrier()
        _agg_blocks(KF // IB, load_idx, h_buf, acc, src_v, dst_v,
                    (b0, b1, b2, b3, b4, b5), (s0, s1, s2, s3, s4, s5))
        plsc.subcore_barrier()
        pltpu.sync_copy(acc.at[pl.ds(r0, RPT)],
                        out_hbm.at[c, pl.ds(r0, RPT), pl.ds(lo, 64)])


_aggf_kernel = functools.partial(
    pl.kernel,
    out_type=jax.ShapeDtypeStruct((NSC, NP, 128), jnp.float32),
    mesh=_MESH,
    scratch_types=[
        pltpu.VMEM_SHARED((NP, 64), jnp.float32),
        pltpu.VMEM_SHARED((NP, 64), jnp.float32),
        pltpu.VMEM((IB, CHUNK), jnp.int32),
        pltpu.VMEM((IB, CHUNK), jnp.int32),
        pltpu.VMEM((HC, 64), jnp.float32),
        pltpu.VMEM((HC, 64), jnp.float32),
        pltpu.VMEM((HC, 64), jnp.float32),
        pltpu.VMEM((HC, 64), jnp.float32),
        pltpu.VMEM((HC, 64), jnp.float32),
        pltpu.VMEM((HC, 64), jnp.float32),
        pltpu.SemaphoreType.DMA,
        pltpu.SemaphoreType.DMA,
        pltpu.SemaphoreType.DMA,
        pltpu.SemaphoreType.DMA,
        pltpu.SemaphoreType.DMA,
        pltpu.SemaphoreType.DMA,
    ],
    compiler_params=_SC_PARAMS,
)(_aggf_body)


def _agge_body(h_hbm, src_hbm, dst_hbm, zeros_hbm, out_hbm, h_buf, acc, src_v,
               dst_v, b0, b1, b2, b3, b4, b5, s0, s1, s2, s3, s4, s5):
    c = lax.axis_index("c")
    s = lax.axis_index("s")
    r0 = s * RPT
    pltpu.sync_copy(h_hbm.at[pl.ds(r0, RPT), pl.ds(0, L3W)],
                    h_buf.at[pl.ds(r0, RPT)])
    pltpu.sync_copy(zeros_hbm.at[pl.ds(r0, RPT), pl.ds(0, L3W)],
                    acc.at[pl.ds(r0, RPT)])
    plsc.subcore_barrier()

    def load_idx(b):
        pltpu.sync_copy(src_hbm.at[c, s, pl.ds(b * IB, IB)], src_v)
        pltpu.sync_copy(dst_hbm.at[c, s, pl.ds(b * IB, IB)], dst_v)

    _agg_blocks(KE // IB, load_idx, h_buf, acc, src_v, dst_v,
                (b0, b1, b2, b3, b4, b5), (s0, s1, s2, s3, s4, s5))
    plsc.subcore_barrier()
    pltpu.sync_copy(acc.at[pl.ds(r0, RPT)],
                    out_hbm.at[c, pl.ds(r0, RPT), pl.ds(0, L3W)])


_agge_kernel = functools.partial(
    pl.kernel,
    out_type=jax.ShapeDtypeStruct((NSC, NP, 128), jnp.float32),
    mesh=_MESH,
    scratch_types=[
        pltpu.VMEM_SHARED((NP, L3W), jnp.float32),
        pltpu.VMEM_SHARED((NP, L3W), jnp.float32),
        pltpu.VMEM((IB, CHUNK), jnp.int32),
        pltpu.VMEM((IB, CHUNK), jnp.int32),
        pltpu.VMEM((HC, L3W), jnp.float32),
        pltpu.VMEM((HC, L3W), jnp.float32),
        pltpu.VMEM((HC, L3W), jnp.float32),
        pltpu.VMEM((HC, L3W), jnp.float32),
        pltpu.VMEM((HC, L3W), jnp.float32),
        pltpu.VMEM((HC, L3W), jnp.float32),
        pltpu.SemaphoreType.DMA,
        pltpu.SemaphoreType.DMA,
        pltpu.SemaphoreType.DMA,
        pltpu.SemaphoreType.DMA,
        pltpu.SemaphoreType.DMA,
        pltpu.SemaphoreType.DMA,
    ],
    compiler_params=_SC_PARAMS,
)(_agge_body)


def _tc_mm1_body(x_ref, w1_ref, h_ref):
    h_ref[...] = jnp.dot(x_ref[...], w1_ref[...],
                         preferred_element_type=jnp.float32)


def _tc1_body(hraw_ref, degp_ref, h_ref, dinv_ref):
    deg = degp_ref[0, 0:N, 0:1] + degp_ref[1, 0:N, 0:1] + 1.0
    dinv = lax.rsqrt(deg)
    sh = hraw_ref[...] * dinv
    h_ref[0:N, :] = sh[:, 0:128]
    h_ref[NP:NP + N, :] = sh[:, 128:256]
    dinv_ref[...] = jnp.broadcast_to(dinv, (N, 8))


def _tc_mid_body(agg_ref, dinv_ref, b_ref, g_ref, be_ref, w_ref, out_ref):
    dinv = dinv_ref[:, 0:1]
    acts = []
    for i in range(2):
        z = agg_ref[i, 0:N, :] * dinv + b_ref[i]
        m = jnp.mean(z, axis=0, keepdims=True)
        zc = z - m
        v = jnp.mean(zc * zc, axis=0, keepdims=True)
        y = zc * lax.rsqrt(v + 1e-5) * g_ref[i] + be_ref[i]
        acts.append(jnp.maximum(y, 0.0))
    for j in range(2):
        hj = (jnp.dot(acts[0], w_ref[0, j], preferred_element_type=jnp.float32)
              + jnp.dot(acts[1], w_ref[1, j], preferred_element_type=jnp.float32))
        out_ref[j * NP:j * NP + N, :] = hj * dinv


def _tc_pre3_body(agg_ref, dinv_ref, b_ref, g_ref, be_ref, w_ref, out_ref):
    dinv = dinv_ref[:, 0:1]
    acts = []
    for i in range(2):
        z = agg_ref[i, 0:N, :] * dinv + b_ref[i]
        m = jnp.mean(z, axis=0, keepdims=True)
        zc = z - m
        v = jnp.mean(zc * zc, axis=0, keepdims=True)
        y = zc * lax.rsqrt(v + 1e-5) * g_ref[i] + be_ref[i]
        acts.append(jnp.maximum(y, 0.0))
    h3 = (jnp.dot(acts[0], w_ref[0], preferred_element_type=jnp.float32)
          + jnp.dot(acts[1], w_ref[1], preferred_element_type=jnp.float32))
    out_ref[0:N, 0:L3W] = h3 * dinv


def _tc_out_body(p_ref, h3_ref, dinv_ref, b3_ref, out_ref):
    t = (p_ref[0, 0:N, 0:C] + p_ref[1, 0:N, 0:C] + h3_ref[0:N, 0:C])
    t = t * dinv_ref[:, 0:1] + b3_ref[...]
    mx = jnp.max(t, axis=1, keepdims=True)
    e = jnp.exp(t - mx)
    lse = jnp.log(jnp.sum(e, axis=1, keepdims=True))
    out_ref[...] = t - mx - lse


def kernel(x, edge_index, relations, W1, b1, g1, be1, W2, b2, g2, be2, W3, b3):
    del relations
    src = edge_index[0].astype(jnp.int32)
    dst = edge_index[1].astype(jnp.int32)
    pad = E_PAD - E
    src_p = jnp.concatenate([src, jnp.zeros((pad,), jnp.int32)])
    dst_p = jnp.concatenate([dst, jnp.full((pad,), N, jnp.int32)])
    src_f = src_p.reshape(NT, KF, CHUNK)
    dst_f = dst_p.reshape(NT, KF, CHUNK)
    src_e = src_p.reshape(NSC, NT, KE, CHUNK)
    dst_e = dst_p.reshape(NSC, NT, KE, CHUNK)

    zeros_np = jnp.zeros((NP, 128), jnp.float32)
    ones_ch = jnp.ones((CHUNK, 128), jnp.float32)
    b1r, g1r, be1r = (a.reshape(2, 1, 128) for a in (b1, g1, be1))
    b2r, g2r, be2r = (a.reshape(2, 1, 128) for a in (b2, g2, be2))
    w2r = W2.reshape(2, 128, 2, 128).transpose(0, 2, 1, 3)
    w3p = jnp.pad(W3, ((0, 0), (0, L3W - C))).reshape(2, 128, L3W)
    b3r = b3.reshape(1, C)

    degp = _deg_kernel(dst_e, zeros_np, ones_ch)
    hraw = pl.pallas_call(
        _tc_mm1_body,
        out_shape=jax.ShapeDtypeStruct((N, HD), jnp.float32),
    )(x, W1)

    h1, dinv8 = pl.pallas_call(
        _tc1_body,
        out_shape=[
            jax.ShapeDtypeStruct((NSC * NP, 128), jnp.float32),
            jax.ShapeDtypeStruct((N, 8), jnp.float32),
        ],
    )(hraw, degp)

    a1 = _aggf_kernel(h1, src_f, dst_f)

    mid = pl.pallas_call(
        _tc_mid_body,
        out_shape=jax.ShapeDtypeStruct((NSC * NP, 128), jnp.float32),
    )
    h2 = mid(a1, dinv8, b1r, g1r, be1r, w2r)

    a2 = _aggf_kernel(h2, src_f, dst_f)

    h3 = pl.pallas_call(
        _tc_pre3_body,
        out_shape=jax.ShapeDtypeStruct((NP, 128), jnp.float32),
    )(a2, dinv8, b2r, g2r, be2r, w3p)

    a3 = _agge_kernel(h3, src_e, dst_e, zeros_np)

    out = pl.pallas_call(
        _tc_out_body,
        out_shape=jax.ShapeDtypeStruct((N, C), jnp.float32),
    )(a3, h3, dinv8, b3r)
    return out


# speedup vs baseline: 1.2401x; 1.2401x over previous
import functools

import jax
import jax.numpy as jnp
from jax import lax
from jax.experimental import pallas as pl
from jax.experimental.pallas import tpu as pltpu
from jax.experimental.pallas import tpu_sc as plsc

N = 10000
E = 320000
F = 128
HD = 256
C = 40
L3W = 48

NSC = 2
NT = 16
CHUNK = 128

NP = N + 112
RPT = NP // NT

E_PAD = 327680
KF = E_PAD // (NT * CHUNK)
KE = E_PAD // (NSC * NT * CHUNK)
IB = 8
HC = 64
NB = 6

_MESH = plsc.VectorSubcoreMesh(core_axis_name="c", subcore_axis_name="s")
_SC_PARAMS = pltpu.CompilerParams(use_tc_tiling_on_sc=False)


def _deg_body(dst_hbm, zeros_hbm, ones_hbm, out_hbm, acc, dst_v, ones_v, sem):
    c = lax.axis_index("c")
    s = lax.axis_index("s")
    r0 = s * RPT
    pltpu.sync_copy(zeros_hbm.at[pl.ds(r0, RPT), pl.ds(0, 8)],
                    acc.at[pl.ds(r0, RPT)])
    pltpu.sync_copy(ones_hbm.at[:, pl.ds(0, 8)], ones_v)
    pltpu.sync_copy(dst_hbm.at[c, s], dst_v)
    plsc.subcore_barrier()

    def step(j, carry):
        pltpu.sync_copy(ones_v, acc.at[dst_v.at[j]], add=True)
        return carry

    lax.fori_loop(0, KE, step, 0)
    plsc.subcore_barrier()
    pltpu.sync_copy(acc.at[pl.ds(r0, RPT)],
                    out_hbm.at[c, pl.ds(r0, RPT), pl.ds(0, 8)])


_deg_kernel = functools.partial(
    pl.kernel,
    out_type=jax.ShapeDtypeStruct((NSC, NP, 128), jnp.float32),
    mesh=_MESH,
    scratch_types=[
        pltpu.VMEM_SHARED((NP, 8), jnp.float32),
        pltpu.VMEM((KE, CHUNK), jnp.int32),
        pltpu.VMEM((CHUNK, 8), jnp.float32),
        pltpu.SemaphoreType.DMA,
    ],
    compiler_params=_SC_PARAMS,
)(_deg_body)


def _agg_blocks(n_blocks, load_idx, h_hbm, acc, src_v, dst_v, bufs, sems):
    nhc = 2 * IB

    def block(b, carry):
        load_idx(b)

        def issue(k):
            j, p = divmod(k, 2)
            return pltpu.async_copy(
                h_hbm.at[src_v.at[j, pl.ds(p * HC, HC)]],
                bufs[k % NB], sems[k % NB])

        descs = [None] * nhc
        for k in range(NB - 1):
            descs[k] = issue(k)
        for k in range(nhc):
            descs[k].wait()
            if k + NB - 1 < nhc:
                descs[k + NB - 1] = issue(k + NB - 1)
            j, p = divmod(k, 2)
            pltpu.sync_copy(bufs[k % NB],
                            acc.at[dst_v.at[j, pl.ds(p * HC, HC)]], add=True)
        return carry

    lax.fori_loop(0, n_blocks, block, 0)


def _aggf_body(h_hbm, src_hbm, dstf_hbm, out_hbm, h_buf, acc, src_v, dst_v,
               b0, b1, b2, b3, b4, b5, s0, s1, s2, s3, s4, s5):
    c = lax.axis_index("c")
    s = lax.axis_index("s")
    r0 = s * RPT

    def load_idx(b):
        pltpu.sync_copy(src_hbm.at[s, pl.ds(b * IB, IB)], src_v)
        pltpu.sync_copy(dstf_hbm.at[s, pl.ds(b * IB, IB)], dst_v)

    for qq in range(2):
        lo = qq * 64
        pltpu.sync_copy(h_hbm.at[pl.ds(c * NP + r0, RPT), pl.ds(lo, 64)],
                        h_buf.at[pl.ds(r0, RPT)])
        pltpu.sync_copy(h_hbm.at[pl.ds(c * NP + r0, RPT), pl.ds(lo, 64)],
                        acc.at[pl.ds(r0, RPT)])
        plsc.subcore_barrier()
        _agg_blocks(KF // IB, load_idx, h_buf, acc, src_v, dst_v,
                    (b0, b1, b2, b3, b4, b5), (s0, s1, s2, s3, s4, s5))
        plsc.subcore_barrier()
        pltpu.sync_copy(acc.at[pl.ds(r0, RPT)],
                        out_hbm.at[c, pl.ds(r0, RPT), pl.ds(lo, 64)])


_aggf_kernel = functools.partial(
    pl.kernel,
    out_type=jax.ShapeDtypeStruct((NSC, NP, 128), jnp.float32),
    mesh=_MESH,
    scratch_types=[
        pltpu.VMEM_SHARED((NP, 64), jnp.float32),
        pltpu.VMEM_SHARED((NP, 64), jnp.float32),
        pltpu.VMEM((IB, CHUNK), jnp.int32),
        pltpu.VMEM((IB, CHUNK), jnp.int32),
        pltpu.VMEM((HC, 64), jnp.float32),
        pltpu.VMEM((HC, 64), jnp.float32),
        pltpu.VMEM((HC, 64), jnp.float32),
        pltpu.VMEM((HC, 64), jnp.float32),
        pltpu.VMEM((HC, 64), jnp.float32),
        pltpu.VMEM((HC, 64), jnp.float32),
        pltpu.SemaphoreType.DMA,
        pltpu.SemaphoreType.DMA,
        pltpu.SemaphoreType.DMA,
        pltpu.SemaphoreType.DMA,
        pltpu.SemaphoreType.DMA,
        pltpu.SemaphoreType.DMA,
    ],
    compiler_params=_SC_PARAMS,
)(_aggf_body)


def _aggx_body(x_hbm, src_hbm, dstf_hbm, out_hbm, h_buf, acc, src_v, dst_v,
               b0, b1, b2, b3, b4, b5, s0, s1, s2, s3, s4, s5):
    c = lax.axis_index("c")
    s = lax.axis_index("s")
    r0 = s * RPT
    lo = c * 64
    pltpu.sync_copy(x_hbm.at[pl.ds(r0, RPT), pl.ds(lo, 64)],
                    h_buf.at[pl.ds(r0, RPT)])
    pltpu.sync_copy(x_hbm.at[pl.ds(r0, RPT), pl.ds(lo, 64)],
                    acc.at[pl.ds(r0, RPT)])
    plsc.subcore_barrier()

    def load_idx(b):
        pltpu.sync_copy(src_hbm.at[s, pl.ds(b * IB, IB)], src_v)
        pltpu.sync_copy(dstf_hbm.at[s, pl.ds(b * IB, IB)], dst_v)

    _agg_blocks(KF // IB, load_idx, h_buf, acc, src_v, dst_v,
                (b0, b1, b2, b3, b4, b5), (s0, s1, s2, s3, s4, s5))
    plsc.subcore_barrier()
    pltpu.sync_copy(acc.at[pl.ds(r0, RPT)],
                    out_hbm.at[c, pl.ds(r0, RPT), pl.ds(0, 64)])


_aggx_kernel = functools.partial(
    pl.kernel,
    out_type=jax.ShapeDtypeStruct((NSC, NP, 128), jnp.float32),
    mesh=_MESH,
    scratch_types=[
        pltpu.VMEM_SHARED((NP, 64), jnp.float32),
        pltpu.VMEM_SHARED((NP, 64), jnp.float32),
        pltpu.VMEM((IB, CHUNK), jnp.int32),
        pltpu.VMEM((IB, CHUNK), jnp.int32),
        pltpu.VMEM((HC, 64), jnp.float32),
        pltpu.VMEM((HC, 64), jnp.float32),
        pltpu.VMEM((HC, 64), jnp.float32),
        pltpu.VMEM((HC, 64), jnp.float32),
        pltpu.VMEM((HC, 64), jnp.float32),
        pltpu.VMEM((HC, 64), jnp.float32),
        pltpu.SemaphoreType.DMA,
        pltpu.SemaphoreType.DMA,
        pltpu.SemaphoreType.DMA,
        pltpu.SemaphoreType.DMA,
        pltpu.SemaphoreType.DMA,
        pltpu.SemaphoreType.DMA,
    ],
    compiler_params=_SC_PARAMS,
)(_aggx_body)


def _agge_body(h_hbm, src_hbm, dst_hbm, zeros_hbm, out_hbm, h_buf, acc, src_v,
               dst_v, b0, b1, b2, b3, b4, b5, s0, s1, s2, s3, s4, s5):
    c = lax.axis_index("c")
    s = lax.axis_index("s")
    r0 = s * RPT
    pltpu.sync_copy(h_hbm.at[pl.ds(r0, RPT), pl.ds(0, L3W)],
                    h_buf.at[pl.ds(r0, RPT)])
    pltpu.sync_copy(zeros_hbm.at[pl.ds(r0, RPT), pl.ds(0, L3W)],
                    acc.at[pl.ds(r0, RPT)])
    plsc.subcore_barrier()

    def load_idx(b):
        pltpu.sync_copy(src_hbm.at[c, s, pl.ds(b * IB, IB)], src_v)
        pltpu.sync_copy(dst_hbm.at[c, s, pl.ds(b * IB, IB)], dst_v)

    _agg_blocks(KE // IB, load_idx, h_buf, acc, src_v, dst_v,
                (b0, b1, b2, b3, b4, b5), (s0, s1, s2, s3, s4, s5))
    plsc.subcore_barrier()
    pltpu.sync_copy(acc.at[pl.ds(r0, RPT)],
                    out_hbm.at[c, pl.ds(r0, RPT), pl.ds(0, L3W)])


_agge_kernel = functools.partial(
    pl.kernel,
    out_type=jax.ShapeDtypeStruct((NSC, NP, 128), jnp.float32),
    mesh=_MESH,
    scratch_types=[
        pltpu.VMEM_SHARED((NP, L3W), jnp.float32),
        pltpu.VMEM_SHARED((NP, L3W), jnp.float32),
        pltpu.VMEM((IB, CHUNK), jnp.int32),
        pltpu.VMEM((IB, CHUNK), jnp.int32),
        pltpu.VMEM((HC, L3W), jnp.float32),
        pltpu.VMEM((HC, L3W), jnp.float32),
        pltpu.VMEM((HC, L3W), jnp.float32),
        pltpu.VMEM((HC, L3W), jnp.float32),
        pltpu.VMEM((HC, L3W), jnp.float32),
        pltpu.VMEM((HC, L3W), jnp.float32),
        pltpu.SemaphoreType.DMA,
        pltpu.SemaphoreType.DMA,
        pltpu.SemaphoreType.DMA,
        pltpu.SemaphoreType.DMA,
        pltpu.SemaphoreType.DMA,
        pltpu.SemaphoreType.DMA,
    ],
    compiler_params=_SC_PARAMS,
)(_agge_body)


def _tc_a_body(x_ref, degp_ref, sx_ref, dinv_ref):
    deg = degp_ref[0, 0:N, 0:1] + degp_ref[1, 0:N, 0:1] + 1.0
    dinv = lax.rsqrt(deg)
    sx_ref[0:N, :] = x_ref[...] * dinv
    dinv_ref[...] = jnp.broadcast_to(dinv, (N, 8))


def _tc_b_body(aggx_ref, dinv_ref, w1_ref, b1_ref, g1_ref, be1_ref, w2_ref,
               out_ref):
    dinv = dinv_ref[:, 0:1]
    z0 = aggx_ref[0, 0:N, 0:64] * dinv
    z1 = aggx_ref[1, 0:N, 0:64] * dinv
    o1 = (jnp.dot(z0, w1_ref[0], preferred_element_type=jnp.float32)
          + jnp.dot(z1, w1_ref[1], preferred_element_type=jnp.float32)
          + b1_ref[...])
    m = jnp.mean(o1, axis=0, keepdims=True)
    zc = o1 - m
    v = jnp.mean(zc * zc, axis=0, keepdims=True)
    y = zc * lax.rsqrt(v + 1e-5) * g1_ref[...] + be1_ref[...]
    a = jnp.maximum(y, 0.0)
    for j in range(2):
        hj = jnp.dot(a, w2_ref[:, j * 128:(j + 1) * 128],
                     preferred_element_type=jnp.float32)
        out_ref[j * NP:j * NP + N, :] = hj * dinv


def _tc_mid_body(agg_ref, dinv_ref, b_ref, g_ref, be_ref, w_ref, out_ref):
    dinv = dinv_ref[:, 0:1]
    acts = []
    for i in range(2):
        z = agg_ref[i, 0:N, :] * dinv + b_ref[i]
        m = jnp.mean(z, axis=0, keepdims=True)
        zc = z - m
        v = jnp.mean(zc * zc, axis=0, keepdims=True)
        y = zc * lax.rsqrt(v + 1e-5) * g_ref[i] + be_ref[i]
        acts.append(jnp.maximum(y, 0.0))
    for j in range(2):
        hj = (jnp.dot(acts[0], w_ref[0, j], preferred_element_type=jnp.float32)
              + jnp.dot(acts[1], w_ref[1, j], preferred_element_type=jnp.float32))
        out_ref[j * NP:j * NP + N, :] = hj * dinv


def _tc_pre3_body(agg_ref, dinv_ref, b_ref, g_ref, be_ref, w_ref, out_ref):
    dinv = dinv_ref[:, 0:1]
    acts = []
    for i in range(2):
        z = agg_ref[i, 0:N, :] * dinv + b_ref[i]
        m = jnp.mean(z, axis=0, keepdims=True)
        zc = z - m
        v = jnp.mean(zc * zc, axis=0, keepdims=True)
        y = zc * lax.rsqrt(v + 1e-5) * g_ref[i] + be_ref[i]
        acts.append(jnp.maximum(y, 0.0))
    h3 = (jnp.dot(acts[0], w_ref[0], preferred_element_type=jnp.float32)
          + jnp.dot(acts[1], w_ref[1], preferred_element_type=jnp.float32))
    out_ref[0:N, 0:L3W] = h3 * dinv


def _tc_out_body(p_ref, h3_ref, dinv_ref, b3_ref, out_ref):
    t = (p_ref[0, 0:N, 0:C] + p_ref[1, 0:N, 0:C] + h3_ref[0:N, 0:C])
    t = t * dinv_ref[:, 0:1] + b3_ref[...]
    mx = jnp.max(t, axis=1, keepdims=True)
    e = jnp.exp(t - mx)
    lse = jnp.log(jnp.sum(e, axis=1, keepdims=True))
    out_ref[...] = t - mx - lse


def kernel(x, edge_index, relations, W1, b1, g1, be1, W2, b2, g2, be2, W3, b3):
    del relations
    src = edge_index[0].astype(jnp.int32)
    dst = edge_index[1].astype(jnp.int32)
    pad = E_PAD - E
    src_p = jnp.concatenate([src, jnp.zeros((pad,), jnp.int32)])
    dst_p = jnp.concatenate([dst, jnp.full((pad,), N, jnp.int32)])
    src_f = src_p.reshape(NT, KF, CHUNK)
    dst_f = dst_p.reshape(NT, KF, CHUNK)
    src_e = src_p.reshape(NSC, NT, KE, CHUNK)
    dst_e = dst_p.reshape(NSC, NT, KE, CHUNK)

    zeros_np = jnp.zeros((NP, 128), jnp.float32)
    ones_ch = jnp.ones((CHUNK, 128), jnp.float32)
    w1r = W1.reshape(2, 64, HD)
    b1r, g1r, be1r = (a.reshape(1, HD) for a in (b1, g1, be1))
    b2r, g2r, be2r = (a.reshape(2, 1, 128) for a in (b2, g2, be2))
    w3p = jnp.pad(W3, ((0, 0), (0, L3W - C))).reshape(2, 128, L3W)
    b3r = b3.reshape(1, C)

    degp = _deg_kernel(dst_e, zeros_np, ones_ch)

    sx, dinv8 = pl.pallas_call(
        _tc_a_body,
        out_shape=[
            jax.ShapeDtypeStruct((NP, 128), jnp.float32),
            jax.ShapeDtypeStruct((N, 8), jnp.float32),
        ],
    )(x, degp)

    ax = _aggx_kernel(sx, src_f, dst_f)

    h2 = pl.pallas_call(
        _tc_b_body,
        out_shape=jax.ShapeDtypeStruct((NSC * NP, 128), jnp.float32),
    )(ax, dinv8, w1r, b1r, g1r, be1r, W2)

    a2 = _aggf_kernel(h2, src_f, dst_f)

    h3 = pl.pallas_call(
        _tc_pre3_body,
        out_shape=jax.ShapeDtypeStruct((NP, 128), jnp.float32),
    )(a2, dinv8, b2r, g2r, be2r, w3p)

    a3 = _agge_kernel(h3, src_e, dst_e, zeros_np)

    out = pl.pallas_call(
        _tc_out_body,
        out_shape=jax.ShapeDtypeStruct((N, C), jnp.float32),
    )(a3, h3, dinv8, b3r)
    return out
